# baseline (speedup 1.0000x reference)
import dataclasses
import functools

import jax
import jax.numpy as jnp
from jax import lax
from jax.experimental import pallas as pl
from jax.experimental.pallas import tpu as pltpu
from jax.experimental.pallas import tpu_sc as plsc

N = 100000
E = 3200000
N_ITERS = 32
NC = 2
NS = 16
NW = NC * NS
EPW = E // NW
CH = 1280
CHP = 640
ZCH = 800
NZCH = N // ZCH
PCH = 5000
L = 16

NP = 100096
NPR = NP // 128

_mesh = plsc.VectorSubcoreMesh(core_axis_name="c", subcore_axis_name="s")

_sc_params = pltpu.CompilerParams()
if "needs_layout_passes" in pltpu.CompilerParams.__dataclass_fields__:
    _sc_params = dataclasses.replace(_sc_params, needs_layout_passes=False)


def _fused_body(bvec, dst2d, src2d, w2d, q, probs, accum, z_s, b_s, z_tile,
                val_v, f0, f1, f2, xsem):
    cid = lax.axis_index("c")
    sid = lax.axis_index("s")
    zeros16 = jnp.zeros((L,), jnp.float32)

    def stage_combine():
        @pl.loop(sid, NZCH, step=NS)
        def _stage(zc):
            off = zc * ZCH
            pltpu.sync_copy(q.at[pl.ds((1 - cid) * N + off, ZCH)], f0)
            pltpu.sync_copy(accum.at[pl.ds(off, ZCH)], f1)
            pltpu.sync_copy(b_s.at[pl.ds(off, ZCH)], f2)

            @pl.loop(0, ZCH, step=L)
            def _(i):
                f0[pl.ds(i, L)] = (
                    f0[pl.ds(i, L)] + f1[pl.ds(i, L)] + f2[pl.ds(i, L)]
                )

            pltpu.sync_copy(f0, z_s.at[pl.ds(off, ZCH)])

    def cross_core_handshake():
        plsc.subcore_barrier()

        @pl.when(sid == 0)
        def _():
            pl.semaphore_signal(xsem, 1, core_index=1 - cid)
            pl.semaphore_wait(xsem, 1)

        plsc.subcore_barrier()

    @pl.loop(sid, NZCH, step=NS)
    def _(zc):
        off = zc * ZCH
        pltpu.sync_copy(bvec.at[pl.ds(off, ZCH)], f0)
        pltpu.sync_copy(f0, b_s.at[pl.ds(off, ZCH)])
        pltpu.sync_copy(f0, z_s.at[pl.ds(off, ZCH)])

    def edge_chunk(dst_v, src_v, w_v):
        @plsc.parallel_loop(0, CH, step=L, unroll=4)
        def _(i):
            zd = plsc.load_gather(z_tile, [dst_v[0, pl.ds(i, L)]])
            val_v[pl.ds(i, L)] = zd * w_v[0, pl.ds(i, L)]

        pltpu.sync_copy(val_v, accum.at[src_v.at[0]], add=True)

    edge_pipeline = pltpu.emit_pipeline(
        edge_chunk,
        grid=(E // CH,),
        in_specs=[
            pl.BlockSpec((1, CH), lambda i: (0, i)),
            pl.BlockSpec((1, CH), lambda i: (0, i)),
            pl.BlockSpec((1, CH), lambda i: (0, i)),
        ],
        core_axis_name=("c", "s"),
        dimension_semantics=(pltpu.PARALLEL,),
    )

    @pl.loop(0, N_ITERS)
    def _(it):
        @pl.when(it > 0)
        def _():
            stage_combine()

        plsc.subcore_barrier()

        @pl.loop(0, ZCH, step=L)
        def _(i):
            f1[pl.ds(i, L)] = zeros16

        @pl.loop(sid, NZCH, step=NS)
        def _(zc):
            pltpu.sync_copy(f1, accum.at[pl.ds(zc * ZCH, ZCH)])

        pltpu.sync_copy(z_s, z_tile)

        plsc.subcore_barrier()

        edge_pipeline(dst2d, src2d, w2d)

        plsc.subcore_barrier()

        @pl.loop(sid, N // PCH, step=NS)
        def _(zc):
            off = zc * PCH
            pltpu.sync_copy(accum.at[pl.ds(off, PCH)],
                            z_tile.at[pl.ds(0, PCH)])
            pltpu.sync_copy(z_tile.at[pl.ds(0, PCH)],
                            q.at[pl.ds(cid * N + off, PCH)])

        cross_core_handshake()

    stage_combine()
    plsc.subcore_barrier()
    pltpu.sync_copy(z_s, z_tile)

    def prob_chunk(dst_v, src_v, w_v, out_v):
        @pl.loop(0, CHP, step=L)
        def _(i):
            zd = plsc.load_gather(z_tile, [dst_v[0, pl.ds(i, L)]])
            zs = plsc.load_gather(z_tile, [src_v[0, pl.ds(i, L)]])
            out_v[0, pl.ds(i, L)] = w_v[0, pl.ds(i, L)] * zd / zs

    pltpu.emit_pipeline(
        prob_chunk,
        grid=(E // CHP,),
        in_specs=[
            pl.BlockSpec((1, CHP), lambda i: (0, i)),
            pl.BlockSpec((1, CHP), lambda i: (0, i)),
            pl.BlockSpec((1, CHP), lambda i: (0, i)),
        ],
        out_specs=[pl.BlockSpec((1, CHP), lambda i: (0, i))],
        core_axis_name=("c", "s"),
        dimension_semantics=(pltpu.PARALLEL,),
    )(dst2d, src2d, w2d, probs)


_fused_kernel = functools.partial(
    pl.kernel,
    out_type=(
        jax.ShapeDtypeStruct((NC * N,), jnp.float32),
        jax.ShapeDtypeStruct((1, E), jnp.float32),
    ),
    mesh=_mesh,
    scratch_types=[
        pltpu.VMEM_SHARED((N,), jnp.float32),
        pltpu.VMEM_SHARED((N,), jnp.float32),
        pltpu.VMEM_SHARED((N,), jnp.float32),
        pltpu.VMEM((N,), jnp.float32),
        pltpu.VMEM((CH,), jnp.float32),
        pltpu.VMEM((ZCH,), jnp.float32),
        pltpu.VMEM((ZCH,), jnp.float32),
        pltpu.VMEM((ZCH,), jnp.float32),
        pltpu.SemaphoreType.REGULAR,
    ],
    compiler_params=_sc_params,
)(_fused_body)


BE = 6400
GE = E // BE


def _enc_body(x_ref, w1_ref, b1_ref, w2_ref, b2_ref, r_ref, lr_ref):
    x = x_ref[...]
    h = jnp.maximum(
        jnp.dot(x, w1_ref[...], preferred_element_type=jnp.float32)
        + b1_ref[...][None, :],
        0.0,
    )
    p = (
        jnp.dot(h, w2_ref[...], preferred_element_type=jnp.float32)
        + b2_ref[...][None, :]
    )
    r = jax.nn.sigmoid(p[:, 0]) * 0.01
    r_ref[...] = r[None, None, :]
    lr_ref[...] = jnp.log(r)[None, None, :]


def _encoder(edge_feats, W1, b1, W2, b2):
    r2d, lr2d = pl.pallas_call(
        _enc_body,
        grid=(GE,),
        in_specs=[
            pl.BlockSpec((BE, 16), lambda i: (i, 0)),
            pl.BlockSpec((16, 64), lambda i: (0, 0)),
            pl.BlockSpec((64,), lambda i: (0,)),
            pl.BlockSpec((64, 1), lambda i: (0, 0)),
            pl.BlockSpec((1,), lambda i: (0,)),
        ],
        out_specs=[
            pl.BlockSpec((1, 1, BE), lambda i: (i, 0, 0)),
            pl.BlockSpec((1, 1, BE), lambda i: (i, 0, 0)),
        ],
        out_shape=[
            jax.ShapeDtypeStruct((GE, 1, BE), jnp.float32),
            jax.ShapeDtypeStruct((GE, 1, BE), jnp.float32),
        ],
    )(edge_feats, W1, b1, W2, b2)
    return r2d.reshape(E), lr2d.reshape(E)


def _comb_body(p0_ref, p1_ref, b_ref, z_ref, lz_ref):
    z = p0_ref[...] + p1_ref[...] + b_ref[...]
    z_ref[...] = z
    lz_ref[...] = jnp.log(z)


def _combine(p0, p1, b):
    pad = NP - N
    p0p = jnp.pad(p0, (0, pad)).reshape(NPR, 128)
    p1p = jnp.pad(p1, (0, pad)).reshape(NPR, 128)
    bp = jnp.pad(b, (0, pad), constant_values=1.0).reshape(NPR, 128)
    z2d, lz2d = pl.pallas_call(
        _comb_body,
        out_shape=[
            jax.ShapeDtypeStruct((NPR, 128), jnp.float32),
            jax.ShapeDtypeStruct((NPR, 128), jnp.float32),
        ],
    )(p0p, p1p, bp)
    return z2d.reshape(NP)[:N], lz2d.reshape(NP)[:N]


def kernel(edge_index, edge_feats, sink_node_mask, W1, b1, W2, b2):
    src = edge_index[0]
    dst = edge_index[1]
    b = sink_node_mask.astype(jnp.float32)

    exp_rewards, log_rewards = _encoder(edge_feats, W1, b1, W2, b2)

    dst2 = dst.reshape(1, E)
    src2 = src.reshape(1, E)
    w2 = exp_rewards.reshape(1, E)

    q, probs2 = _fused_kernel(b, dst2, src2, w2)
    _, log_z = _combine(q[:N], q[N:], b)
    edge_probs = probs2.reshape(E)
    return (log_rewards, log_z, edge_probs)


# speedup vs baseline: 1.6659x; 1.6659x over previous
import dataclasses
import functools

import jax
import jax.numpy as jnp
from jax import lax
from jax.experimental import pallas as pl
from jax.experimental.pallas import tpu as pltpu
from jax.experimental.pallas import tpu_sc as plsc

N = 100000
E = 3200000
N_ITERS = 32
NC = 2
NS = 16
NW = NC * NS
CH = 1280
CHP = 640
ZCH = 800
NZCH = N // ZCH
PCH = 5000
L = 16

NP = 100096
NPR = NP // 128

_mesh = plsc.VectorSubcoreMesh(core_axis_name="c", subcore_axis_name="s")

_sc_params = pltpu.CompilerParams()
if "needs_layout_passes" in pltpu.CompilerParams.__dataclass_fields__:
    _sc_params = dataclasses.replace(_sc_params, needs_layout_passes=False)


def _fused_body(bvec, dst2d, src2d, w2d, q, probs, accum, z_s, b_s, z_tile,
                val_v, f0, f1, f2, xsem):
    cid = lax.axis_index("c")
    sid = lax.axis_index("s")
    zeros16 = jnp.zeros((L,), jnp.float32)

    def stage_combine():
        @pl.loop(sid, NZCH, step=NS)
        def _stage(zc):
            off = zc * ZCH
            pltpu.sync_copy(q.at[pl.ds((1 - cid) * N + off, ZCH)], f0)
            pltpu.sync_copy(accum.at[pl.ds(off, ZCH)], f1)
            pltpu.sync_copy(b_s.at[pl.ds(off, ZCH)], f2)

            @pl.loop(0, ZCH, step=L)
            def _(i):
                f0[pl.ds(i, L)] = (
                    f0[pl.ds(i, L)] + f1[pl.ds(i, L)] + f2[pl.ds(i, L)]
                )

            pltpu.sync_copy(f0, z_s.at[pl.ds(off, ZCH)])

    def cross_core_handshake():
        plsc.subcore_barrier()

        @pl.when(sid == 0)
        def _():
            pl.semaphore_signal(xsem, 1, core_index=1 - cid)
            pl.semaphore_wait(xsem, 1)

        plsc.subcore_barrier()

    @pl.loop(sid, NZCH, step=NS)
    def _(zc):
        off = zc * ZCH
        pltpu.sync_copy(bvec.at[pl.ds(off, ZCH)], f0)
        pltpu.sync_copy(f0, b_s.at[pl.ds(off, ZCH)])
        pltpu.sync_copy(f0, z_s.at[pl.ds(off, ZCH)])

    def edge_chunk(dst_v, src_v, w_v):
        @plsc.parallel_loop(0, CH, step=L, unroll=4)
        def _(i):
            zd = plsc.load_gather(z_tile, [dst_v[0, pl.ds(i, L)]])
            val_v[pl.ds(i, L)] = zd * w_v[0, pl.ds(i, L)]

        pltpu.sync_copy(val_v, accum.at[src_v.at[0]], add=True)

    edge_pipeline = pltpu.emit_pipeline(
        edge_chunk,
        grid=(E // CH,),
        in_specs=[
            pl.BlockSpec((1, CH), lambda i: (0, i)),
            pl.BlockSpec((1, CH), lambda i: (0, i)),
            pl.BlockSpec((1, CH), lambda i: (0, i)),
        ],
        core_axis_name=("c", "s"),
        dimension_semantics=(pltpu.PARALLEL,),
    )

    @pl.loop(0, N_ITERS)
    def _(it):
        @pl.when(it > 0)
        def _():
            stage_combine()

        plsc.subcore_barrier()

        @pl.loop(0, ZCH, step=L)
        def _(i):
            f1[pl.ds(i, L)] = zeros16

        @pl.loop(sid, NZCH, step=NS)
        def _(zc):
            pltpu.sync_copy(f1, accum.at[pl.ds(zc * ZCH, ZCH)])

        pltpu.sync_copy(z_s, z_tile)

        plsc.subcore_barrier()

        edge_pipeline(dst2d, src2d, w2d)

        plsc.subcore_barrier()

        @pl.loop(sid, N // PCH, step=NS)
        def _(zc):
            off = zc * PCH
            pltpu.sync_copy(accum.at[pl.ds(off, PCH)],
                            z_tile.at[pl.ds(0, PCH)])
            pltpu.sync_copy(z_tile.at[pl.ds(0, PCH)],
                            q.at[pl.ds(cid * N + off, PCH)])

        cross_core_handshake()

    stage_combine()
    plsc.subcore_barrier()
    pltpu.sync_copy(z_s, z_tile)

    def prob_chunk(dst_v, src_v, w_v, out_v):
        @pl.loop(0, CHP, step=L)
        def _(i):
            zd = plsc.load_gather(z_tile, [dst_v[0, pl.ds(i, L)]])
            zs = plsc.load_gather(z_tile, [src_v[0, pl.ds(i, L)]])
            out_v[0, pl.ds(i, L)] = w_v[0, pl.ds(i, L)] * zd / zs

    pltpu.emit_pipeline(
        prob_chunk,
        grid=(E // CHP,),
        in_specs=[
            pl.BlockSpec((1, CHP), lambda i: (0, i)),
            pl.BlockSpec((1, CHP), lambda i: (0, i)),
            pl.BlockSpec((1, CHP), lambda i: (0, i)),
        ],
        out_specs=[pl.BlockSpec((1, CHP), lambda i: (0, i))],
        core_axis_name=("c", "s"),
        dimension_semantics=(pltpu.PARALLEL,),
    )(dst2d, src2d, w2d, probs)


_fused_kernel = functools.partial(
    pl.kernel,
    out_type=(
        jax.ShapeDtypeStruct((NC * N,), jnp.float32),
        jax.ShapeDtypeStruct((1, E), jnp.float32),
    ),
    mesh=_mesh,
    scratch_types=[
        pltpu.VMEM_SHARED((N,), jnp.float32),
        pltpu.VMEM_SHARED((N,), jnp.float32),
        pltpu.VMEM_SHARED((N,), jnp.float32),
        pltpu.VMEM((N,), jnp.float32),
        pltpu.VMEM((CH,), jnp.float32),
        pltpu.VMEM((ZCH,), jnp.float32),
        pltpu.VMEM((ZCH,), jnp.float32),
        pltpu.VMEM((ZCH,), jnp.float32),
        pltpu.SemaphoreType.REGULAR,
    ],
    compiler_params=_sc_params,
)(_fused_body)


BE = 6400
GE = E // BE


def _enc_body(x_ref, w1_ref, b1_ref, w2_ref, b2_ref, r_ref, lr_ref):
    x = x_ref[...]
    h = jnp.maximum(
        jnp.dot(x, w1_ref[...], preferred_element_type=jnp.float32)
        + b1_ref[...][None, :],
        0.0,
    )
    p = (
        jnp.dot(h, w2_ref[...], preferred_element_type=jnp.float32)
        + b2_ref[...][None, :]
    )
    r = jax.nn.sigmoid(p[:, 0]) * 0.01
    r_ref[...] = r[None, None, :]
    lr_ref[...] = jnp.log(r)[None, None, :]


def _encoder(edge_feats, W1, b1, W2, b2):
    r2d, lr2d = pl.pallas_call(
        _enc_body,
        grid=(GE,),
        in_specs=[
            pl.BlockSpec((BE, 16), lambda i: (i, 0)),
            pl.BlockSpec((16, 64), lambda i: (0, 0)),
            pl.BlockSpec((64,), lambda i: (0,)),
            pl.BlockSpec((64, 1), lambda i: (0, 0)),
            pl.BlockSpec((1,), lambda i: (0,)),
        ],
        out_specs=[
            pl.BlockSpec((1, 1, BE), lambda i: (i, 0, 0)),
            pl.BlockSpec((1, 1, BE), lambda i: (i, 0, 0)),
        ],
        out_shape=[
            jax.ShapeDtypeStruct((GE, 1, BE), jnp.float32),
            jax.ShapeDtypeStruct((GE, 1, BE), jnp.float32),
        ],
    )(edge_feats, W1, b1, W2, b2)
    return r2d.reshape(E), lr2d.reshape(E)


def _comb_body(p0_ref, p1_ref, b_ref, z_ref, lz_ref):
    z = p0_ref[...] + p1_ref[...] + b_ref[...]
    z_ref[...] = z
    lz_ref[...] = jnp.log(z)


def _combine(p0, p1, b):
    pad = NP - N
    p0p = jnp.pad(p0, (0, pad)).reshape(NPR, 128)
    p1p = jnp.pad(p1, (0, pad)).reshape(NPR, 128)
    bp = jnp.pad(b, (0, pad), constant_values=1.0).reshape(NPR, 128)
    z2d, lz2d = pl.pallas_call(
        _comb_body,
        out_shape=[
            jax.ShapeDtypeStruct((NPR, 128), jnp.float32),
            jax.ShapeDtypeStruct((NPR, 128), jnp.float32),
        ],
    )(p0p, p1p, bp)
    return z2d.reshape(NP)[:N], lz2d.reshape(NP)[:N]


def kernel(edge_index, edge_feats, sink_node_mask, W1, b1, W2, b2):
    src = edge_index[0]
    dst = edge_index[1]
    b = sink_node_mask.astype(jnp.float32)

    exp_rewards, log_rewards = _encoder(edge_feats, W1, b1, W2, b2)

    dst2 = dst.reshape(1, E)
    src2 = src.reshape(1, E)
    w2 = exp_rewards.reshape(1, E)

    q, probs2 = _fused_kernel(b, dst2, src2, w2)
    _, log_z = _combine(q[:N], q[N:], b)
    edge_probs = probs2.reshape(E)
    return (log_rewards, log_z, edge_probs)
